# baseline (speedup 1.0000x reference)
import jax
import jax.numpy as jnp
from jax import lax
from jax.experimental import pallas as pl
from jax.experimental.pallas import tpu as pltpu
from jax.experimental.pallas import tpu_sc as plsc

B = 16
N = 512 * 512
K = 16384
EMAX = 8.0
SCALE = (K - 1) / EMAX
NTILES = 32
ROWS_PER_TILE = 256
CROWS = 16
NCH = ROWS_PER_TILE // CROWS
VECS = CROWS * 512 // 16
KR, KC = 128, 128


def _sc_hist(x_hbm, t_hbm, opk_hbm, xb0, xb1, tb0, tb1, hp,
             sx0, sx1, st0, st1):
    cid = lax.axis_index("c")
    sid = lax.axis_index("s")
    wid = sid * 2 + cid
    img = wid // 2
    r0 = (wid % 2) * ROWS_PER_TILE

    zi = jnp.zeros((16,), jnp.int32)

    @plsc.parallel_loop(0, K // 16, unroll=8)
    def _(i):
        hp[pl.ds(i * 16, 16)] = zi

    xbufs, tbufs = (xb0, xb1), (tb0, tb1)
    sxs, sts = (sx0, sx1), (st0, st1)

    def chunk_copies(c, p):
        rr = pl.ds(r0 + c * CROWS, CROWS)
        return (
            pltpu.make_async_copy(x_hbm.at[img, 0, rr, :], xbufs[p], sxs[p]),
            pltpu.make_async_copy(t_hbm.at[img, 0, rr, :], tbufs[p], sts[p]),
        )

    for p in range(2):
        for h in chunk_copies(p, p):
            h.start()

    def pair_body(j, carry):
        for p in range(2):
            c = 2 * j + p
            for h in chunk_copies(c, p):
                h.wait()
            xbuf, tbuf = xbufs[p], tbufs[p]

            @plsc.parallel_loop(0, VECS, unroll=4)
            def _(i):
                r = lax.shift_right_logical(i, 5)
                k = jnp.bitwise_and(i, 31)
                xv = xbuf[r, pl.ds(k * 16, 16)]
                tv = tbuf[r, pl.ds(k * 16, 16)]
                xs = xv * SCALE
                f2 = (SCALE + 1.0) - xs * (2.0 * tv - 1.0)
                fc = jnp.minimum(f2, float(K - 1))
                idx = jnp.maximum(fc.astype(jnp.int32), 0)
                pk = jnp.left_shift(tv.astype(jnp.int32), 16) + 1
                plsc.addupdate_scatter(hp, [idx], pk)

            @pl.when(j < NCH // 2 - 1)
            def _():
                for h in chunk_copies(c + 2, p):
                    h.start()
        return carry

    lax.fori_loop(0, NCH // 2, pair_body, 0)

    pltpu.sync_copy(hp, opk_hbm.at[pl.ds(wid * K, K)])


_phase1 = pl.kernel(
    _sc_hist,
    out_type=jax.ShapeDtypeStruct((NTILES * K,), jnp.int32),
    mesh=plsc.VectorSubcoreMesh(core_axis_name="c", subcore_axis_name="s"),
    compiler_params=pltpu.CompilerParams(needs_layout_passes=False),
    scratch_types=[
        pltpu.VMEM((CROWS, 512), jnp.float32),
        pltpu.VMEM((CROWS, 512), jnp.float32),
        pltpu.VMEM((CROWS, 512), jnp.float32),
        pltpu.VMEM((CROWS, 512), jnp.float32),
        pltpu.VMEM((K,), jnp.int32),
        pltpu.SemaphoreType.DMA,
        pltpu.SemaphoreType.DMA,
        pltpu.SemaphoreType.DMA,
        pltpu.SemaphoreType.DMA,
    ],
)


def _tc_finish(hp_ref, o_ref):
    pk = hp_ref[:, 0] + hp_ref[:, 1]
    n3 = jnp.bitwise_and(pk, 0xFFFF).astype(jnp.float32)
    s3 = jnp.right_shift(pk, 16).astype(jnp.float32)

    br = lax.broadcasted_iota(jnp.int32, (KR, KC), 0)
    bc = lax.broadcasted_iota(jnp.int32, (KR, KC), 1)
    bidx = br * KC + bc
    centers = jnp.where(bidx == 0, 0.0,
                        (bidx.astype(jnp.float32) - 0.5) * (1.0 / SCALE))
    w3 = n3 * centers[None]

    r = lax.broadcasted_iota(jnp.int32, (KC, KC), 0)
    c = lax.broadcasted_iota(jnp.int32, (KC, KC), 1)
    upper = (r <= c).astype(jnp.float32)
    strict = (r < c).astype(jnp.float32)

    incl_n = jnp.dot(n3.reshape(B * KR, KC), upper,
                     preferred_element_type=jnp.float32).reshape(B, KR, KC)
    incl_s = jnp.dot(s3.reshape(B * KR, KC), upper,
                     preferred_element_type=jnp.float32).reshape(B, KR, KC)
    rt_n = jnp.sum(n3, axis=2)
    rt_s = jnp.sum(s3, axis=2)
    prev_n = jnp.dot(rt_n, strict, preferred_element_type=jnp.float32)
    prev_s = jnp.dot(rt_s, strict, preferred_element_type=jnp.float32)
    incl_n = incl_n + prev_n[:, :, None]
    incl_s = incl_s + prev_s[:, :, None]

    tot_n = jnp.sum(rt_n, axis=1)[:, None, None]
    g = jnp.sum(rt_s, axis=1)[:, None, None]
    m_above = tot_n - incl_n
    s_above = g - incl_s
    d1 = g + m_above - s_above
    d2 = d1 + n3 - s3
    num = n3 * (g - s_above) + m_above * s3
    dj = jnp.where(
        d1 > 0.0,
        num / jnp.maximum(d1 * d2, 1.0),
        (m_above + n3) / jnp.maximum(d2, 1.0),
    )
    o_ref[0, 0] = jnp.sum(w3 * dj / jnp.maximum(n3, 1.0)) * (1.0 / B)


def _phase2(hpk):
    return pl.pallas_call(
        _tc_finish,
        in_specs=[
            pl.BlockSpec((B, 2, KR, KC), lambda: (0, 0, 0, 0)),
        ],
        out_specs=pl.BlockSpec(
            (1, 1), lambda: (0, 0), memory_space=pltpu.SMEM
        ),
        out_shape=jax.ShapeDtypeStruct((1, 1), jnp.float32),
    )(hpk)


def kernel(input, target):
    hpk = _phase1(input, target)
    out = _phase2(hpk.reshape(B, 2, KR, KC))
    return out[0, 0]


# speedup vs baseline: 1.0884x; 1.0884x over previous
import jax
import jax.numpy as jnp
from jax import lax
from jax.experimental import pallas as pl
from jax.experimental.pallas import tpu as pltpu
from jax.experimental.pallas import tpu_sc as plsc

B = 16
N = 512 * 512
K = 16384
EMAX = 8.0
SCALE = (K - 1) / EMAX
NTILES = 32
ROWS_PER_TILE = 256
CROWS = 16
NCH = ROWS_PER_TILE // CROWS
VECS = CROWS * 512 // 16
KR, KC = 128, 128


def _sc_hist(x_hbm, t_hbm, opk_hbm, xb0, xb1, tb0, tb1, hp,
             sx0, sx1, st0, st1):
    cid = lax.axis_index("c")
    sid = lax.axis_index("s")
    wid = sid * 2 + cid
    img = wid // 2
    r0 = (wid % 2) * ROWS_PER_TILE

    zi = jnp.zeros((16,), jnp.int32)

    @plsc.parallel_loop(0, K // 16, unroll=8)
    def _(i):
        hp[pl.ds(i * 16, 16)] = zi

    xbufs, tbufs = (xb0, xb1), (tb0, tb1)
    sxs, sts = (sx0, sx1), (st0, st1)

    def chunk_copies(c, p):
        rr = pl.ds(r0 + c * CROWS, CROWS)
        return (
            pltpu.make_async_copy(x_hbm.at[img, 0, rr, :], xbufs[p], sxs[p]),
            pltpu.make_async_copy(t_hbm.at[img, 0, rr, :], tbufs[p], sts[p]),
        )

    for p in range(2):
        for h in chunk_copies(p, p):
            h.start()

    def pair_body(j, carry):
        for p in range(2):
            c = 2 * j + p
            for h in chunk_copies(c, p):
                h.wait()
            xbuf, tbuf = xbufs[p], tbufs[p]

            @plsc.parallel_loop(0, VECS, unroll=8)
            def _(i):
                r = lax.shift_right_logical(i, 5)
                k = jnp.bitwise_and(i, 31)
                xv = xbuf[r, pl.ds(k * 16, 16)]
                tv = tbuf[r, pl.ds(k * 16, 16)]
                xs = xv * SCALE
                f2 = (SCALE + 1.0) - xs * (2.0 * tv - 1.0)
                fc = jnp.minimum(f2, float(K - 1))
                idx = jnp.maximum(fc.astype(jnp.int32), 0)
                pk = jnp.left_shift(tv.astype(jnp.int32), 16) + 1
                plsc.addupdate_scatter(hp, [idx], pk)

            @pl.when(j < NCH // 2 - 1)
            def _():
                for h in chunk_copies(c + 2, p):
                    h.start()
        return carry

    lax.fori_loop(0, NCH // 2, pair_body, 0)

    pltpu.sync_copy(hp, opk_hbm.at[pl.ds(wid * K, K)])


_phase1 = pl.kernel(
    _sc_hist,
    out_type=jax.ShapeDtypeStruct((NTILES * K,), jnp.int32),
    mesh=plsc.VectorSubcoreMesh(core_axis_name="c", subcore_axis_name="s"),
    compiler_params=pltpu.CompilerParams(needs_layout_passes=False),
    scratch_types=[
        pltpu.VMEM((CROWS, 512), jnp.float32),
        pltpu.VMEM((CROWS, 512), jnp.float32),
        pltpu.VMEM((CROWS, 512), jnp.float32),
        pltpu.VMEM((CROWS, 512), jnp.float32),
        pltpu.VMEM((K,), jnp.int32),
        pltpu.SemaphoreType.DMA,
        pltpu.SemaphoreType.DMA,
        pltpu.SemaphoreType.DMA,
        pltpu.SemaphoreType.DMA,
    ],
)


def _tc_finish(hp_ref, o_ref):
    pk = hp_ref[:, 0] + hp_ref[:, 1]
    n3 = jnp.bitwise_and(pk, 0xFFFF).astype(jnp.float32)
    s3 = jnp.right_shift(pk, 16).astype(jnp.float32)

    br = lax.broadcasted_iota(jnp.int32, (KR, KC), 0)
    bc = lax.broadcasted_iota(jnp.int32, (KR, KC), 1)
    bidx = br * KC + bc
    centers = jnp.where(bidx == 0, 0.0,
                        (bidx.astype(jnp.float32) - 0.5) * (1.0 / SCALE))
    w3 = n3 * centers[None]

    r = lax.broadcasted_iota(jnp.int32, (KC, KC), 0)
    c = lax.broadcasted_iota(jnp.int32, (KC, KC), 1)
    upper = (r <= c).astype(jnp.float32)
    strict = (r < c).astype(jnp.float32)

    incl_n = jnp.dot(n3.reshape(B * KR, KC), upper,
                     preferred_element_type=jnp.float32).reshape(B, KR, KC)
    incl_s = jnp.dot(s3.reshape(B * KR, KC), upper,
                     preferred_element_type=jnp.float32).reshape(B, KR, KC)
    rt_n = jnp.sum(n3, axis=2)
    rt_s = jnp.sum(s3, axis=2)
    prev_n = jnp.dot(rt_n, strict, preferred_element_type=jnp.float32)
    prev_s = jnp.dot(rt_s, strict, preferred_element_type=jnp.float32)
    incl_n = incl_n + prev_n[:, :, None]
    incl_s = incl_s + prev_s[:, :, None]

    tot_n = jnp.sum(rt_n, axis=1)[:, None, None]
    g = jnp.sum(rt_s, axis=1)[:, None, None]
    m_above = tot_n - incl_n
    s_above = g - incl_s
    d1 = g + m_above - s_above
    d2 = d1 + n3 - s3
    num = n3 * (g - s_above) + m_above * s3
    dj = jnp.where(
        d1 > 0.0,
        num / jnp.maximum(d1 * d2, 1.0),
        (m_above + n3) / jnp.maximum(d2, 1.0),
    )
    o_ref[0, 0] = jnp.sum(w3 * dj / jnp.maximum(n3, 1.0)) * (1.0 / B)


def _phase2(hpk):
    return pl.pallas_call(
        _tc_finish,
        in_specs=[
            pl.BlockSpec((B, 2, KR, KC), lambda: (0, 0, 0, 0)),
        ],
        out_specs=pl.BlockSpec(
            (1, 1), lambda: (0, 0), memory_space=pltpu.SMEM
        ),
        out_shape=jax.ShapeDtypeStruct((1, 1), jnp.float32),
    )(hpk)


def kernel(input, target):
    hpk = _phase1(input, target)
    out = _phase2(hpk.reshape(B, 2, KR, KC))
    return out[0, 0]
